# baseline (speedup 1.0000x reference)
import functools
import numpy as np
import jax
import jax.numpy as jnp
from jax import lax
from jax.experimental import pallas as pl
from jax.experimental.pallas import tpu as pltpu
from jax.experimental.pallas import tpu_sc as plsc

_N_ROWS = 16384
_N_COLS = 1024
_R1 = 5461
_R2 = 10922

_NC = 2
_NS = 16
_GPC = 4
_GC = 128
_RPW = _N_ROWS // 4
_CHUNK = 256

_INT_MIN = np.int32(-(2**31))
_LOW31 = np.int32(2**31 - 1)

_QLEN = np.float32((1.0 - (-1.0)) / 3)
_OFFS = [np.float32(-1.0 + ((1.0 - (-1.0)) / 3) * i) for i in range(3)]


def _sc_quantiles(x):
    mesh = plsc.VectorSubcoreMesh(core_axis_name="c", subcore_axis_name="s")

    @functools.partial(
        pl.kernel,
        mesh=mesh,
        compiler_params=pltpu.CompilerParams(needs_layout_passes=False),
        out_type=[
            jax.ShapeDtypeStruct((_N_COLS,), jnp.float32),
            jax.ShapeDtypeStruct((_N_COLS,), jnp.float32),
        ],
        scratch_types=[
            pltpu.VMEM((_CHUNK, _GC), jnp.float32),
            pltpu.VMEM((256, _GC), jnp.int32),
            pltpu.VMEM((64, _GC), jnp.int32),
            pltpu.VMEM((_GC,), jnp.float32),
            pltpu.VMEM_SHARED((_NS, 256, _GC), jnp.int32),
        ],
    )
    def run(x_hbm, q1_hbm, q2_hbm, buf, hist, rb, qbuf, shared):
        c = lax.axis_index("c")
        s = lax.axis_index("s")
        g_local = s // _GPC
        quarter = s % _GPC
        col0 = (c * _GPC + g_local) * _GC
        row_base = quarter * _RPW
        lane = lax.iota(jnp.int32, 16)
        zeros16 = jnp.zeros((16,), jnp.int32)
        lanecols = [lane + np.int32(h * 16) for h in range(_GC // 16)]
        nh = _GC // 16

        prefix = [[zeros16] * nh, [zeros16] * nh]
        target = [
            [jnp.full((16,), np.int32(_R1 + 1), jnp.int32)] * nh,
            [jnp.full((16,), np.int32(_R2 + 1), jnp.int32)] * nh,
        ]
        prefix = [list(t) for t in prefix]
        target = [list(t) for t in target]

        for p in range(4):
            shift = 24 - 8 * p

            def zero_body(b, _):
                for h in range(nh):
                    hist[b, pl.ds(h * 16, 16)] = zeros16
                return 0

            lax.fori_loop(0, 256, zero_body, 0)

            pref1 = list(prefix[0])
            pref2 = list(prefix[1])

            def chunk_body(ci, _, _p=p, _shift=shift, _p1=pref1, _p2=pref2):
                r0 = row_base + ci * _CHUNK
                pltpu.sync_copy(
                    x_hbm.at[pl.ds(r0, _CHUNK), pl.ds(col0, _GC)], buf)

                def row_body(r, _):
                    for h in range(nh):
                        v = buf[r, pl.ds(h * 16, 16)]
                        i32 = lax.bitcast_convert_type(v, jnp.int32)
                        sk = jnp.where(i32 < 0, i32 ^ _LOW31, i32)
                        ub = sk ^ _INT_MIN
                        bucket = (
                            lax.shift_right_logical(ub, np.int32(_shift))
                            & np.int32(0xFF))
                        if _p == 0:
                            val = jnp.full(
                                (16,), np.int32(0x10001), jnp.int32)
                        else:
                            hi = lax.shift_right_logical(
                                ub, np.int32(32 - 8 * _p))
                            val = (
                                jnp.where(hi == _p1[h],
                                          np.int32(1), np.int32(0))
                                + jnp.where(hi == _p2[h],
                                            np.int32(0x10000), np.int32(0)))
                        plsc.addupdate_scatter(
                            hist, [bucket, lanecols[h]], val)
                    return 0

                lax.fori_loop(0, _CHUNK, row_body, 0)
                return 0

            lax.fori_loop(0, _RPW // _CHUNK, chunk_body, 0)

            pltpu.sync_copy(hist, shared.at[s])
            plsc.subcore_barrier()
            for qq in range(_GPC):
                partner = g_local * _GPC + qq
                for part in range(4):
                    pltpu.sync_copy(
                        shared.at[partner, pl.ds(part * 64, 64)], rb)

                    def add_body(b, _, _part=part, _qq=qq):
                        for h in range(nh):
                            if _qq == 0:
                                hist[_part * 64 + b, pl.ds(h * 16, 16)] = (
                                    rb[b, pl.ds(h * 16, 16)])
                            else:
                                hist[_part * 64 + b, pl.ds(h * 16, 16)] = (
                                    hist[_part * 64 + b, pl.ds(h * 16, 16)]
                                    + rb[b, pl.ds(h * 16, 16)])
                        return 0

                    lax.fori_loop(0, 64, add_body, 0)
            plsc.subcore_barrier()

            for h in range(nh):

                def scan_body(b, carry, _h=h):
                    (run1, sel1, below1, fnd1,
                     run2, sel2, below2, fnd2) = carry
                    cnt = hist[b, pl.ds(_h * 16, 16)]
                    c1 = cnt & np.int32(0xFFFF)
                    c2 = lax.shift_right_logical(cnt, np.int32(16))
                    bvec = zeros16 + b

                    n1 = run1 + c1
                    cr1 = (fnd1 == 0) & (n1 >= target[0][_h])
                    sel1 = jnp.where(cr1, bvec, sel1)
                    below1 = jnp.where(cr1, run1, below1)
                    fnd1 = jnp.where(cr1, zeros16 + 1, fnd1)

                    n2 = run2 + c2
                    cr2 = (fnd2 == 0) & (n2 >= target[1][_h])
                    sel2 = jnp.where(cr2, bvec, sel2)
                    below2 = jnp.where(cr2, run2, below2)
                    fnd2 = jnp.where(cr2, zeros16 + 1, fnd2)
                    return (n1, sel1, below1, fnd1, n2, sel2, below2, fnd2)

                (_, sel1, below1, _, _, sel2, below2, _) = lax.fori_loop(
                    0, 256, scan_body, (zeros16,) * 8)

                prefix[0][h] = (prefix[0][h] << 8) | sel1
                target[0][h] = target[0][h] - below1
                prefix[1][h] = (prefix[1][h] << 8) | sel2
                target[1][h] = target[1][h] - below2

        @pl.when(quarter == 0)
        def _():
            for k, out_hbm in ((0, q1_hbm), (1, q2_hbm)):
                for h in range(nh):
                    sk = prefix[k][h] ^ _INT_MIN
                    ib = jnp.where(sk < 0, sk ^ _LOW31, sk)
                    qbuf[pl.ds(h * 16, 16)] = lax.bitcast_convert_type(
                        ib, jnp.float32)
                pltpu.sync_copy(qbuf, out_hbm.at[pl.ds(col0, _GC)])

    return run(x)


def _tc_remap_body(x_ref, q1_ref, q2_ref, o_ref):
    x = x_ref[...]
    q1 = q1_ref[...].reshape(1, -1)
    q2 = q2_ref[...].reshape(1, -1)
    q0 = jnp.min(x, axis=0, keepdims=True)
    q3 = jnp.max(x, axis=0, keepdims=True)

    def scale(lo, hi):
        itv = hi - lo
        safe = jnp.where(itv == 0.0, np.float32(1.0), itv)
        return jnp.where(itv == 0.0, np.float32(0.0), _QLEN / safe)

    s0 = scale(q0, q1)
    s1 = scale(q1, q2)
    s2 = scale(q2, q3)
    out = jnp.where(
        x < q1,
        _OFFS[0] + x * s0,
        jnp.where(
            (x > q1) & (x < q2),
            _OFFS[1] + x * s1,
            jnp.where(x > q2, _OFFS[2] + x * s2, np.float32(0.0)),
        ),
    )
    o_ref[...] = out


def kernel(inputs):
    q1, q2 = _sc_quantiles(inputs)
    bc = 128
    out = pl.pallas_call(
        _tc_remap_body,
        grid=(_N_COLS // bc,),
        in_specs=[
            pl.BlockSpec((_N_ROWS, bc), lambda j: (0, j)),
            pl.BlockSpec((1, 1, bc), lambda j: (j, 0, 0)),
            pl.BlockSpec((1, 1, bc), lambda j: (j, 0, 0)),
        ],
        out_specs=pl.BlockSpec((_N_ROWS, bc), lambda j: (0, j)),
        out_shape=jax.ShapeDtypeStruct((_N_ROWS, _N_COLS), jnp.float32),
    )(inputs, q1.reshape(8, 1, bc), q2.reshape(8, 1, bc))
    return out


# speedup vs baseline: 3.0021x; 3.0021x over previous
import functools
import numpy as np
import jax
import jax.numpy as jnp
from jax import lax
from jax.experimental import pallas as pl
from jax.experimental.pallas import tpu as pltpu
from jax.experimental.pallas import tpu_sc as plsc

_N_ROWS = 16384
_N_COLS = 1024
_R1 = 5461
_R2 = 10922

_NC = 2
_NS = 16
_GPC = 4
_GC = 128
_RPW = _N_ROWS // 4
_CHUNK = 128
_NH = _GC // 16

_INT_MIN = np.int32(-(2**31))
_LOW31 = np.int32(2**31 - 1)

_QLEN = np.float32((1.0 - (-1.0)) / 3)
_OFFS = [np.float32(-1.0 + ((1.0 - (-1.0)) / 3) * i) for i in range(3)]


def _sc_quantiles(x):
    mesh = plsc.VectorSubcoreMesh(core_axis_name="c", subcore_axis_name="s")

    @functools.partial(
        pl.kernel,
        mesh=mesh,
        compiler_params=pltpu.CompilerParams(needs_layout_passes=False),
        out_type=[
            jax.ShapeDtypeStruct((_N_COLS,), jnp.float32),
            jax.ShapeDtypeStruct((_N_COLS,), jnp.float32),
        ],
        scratch_types=[
            pltpu.VMEM((_CHUNK, _GC), jnp.float32),
            pltpu.VMEM((_CHUNK, _GC), jnp.float32),
            pltpu.VMEM((256, _GC), jnp.int32),
            pltpu.VMEM((64, _GC), jnp.int32),
            pltpu.VMEM((_GC,), jnp.float32),
            pltpu.VMEM_SHARED((_NS, 256, _GC), jnp.int32),
            pltpu.SemaphoreType.DMA,
            pltpu.SemaphoreType.DMA,
        ],
    )
    def run(x_hbm, q1_hbm, q2_hbm, buf0, buf1, hist, rb, qbuf, shared,
            sem0, sem1):
        c = lax.axis_index("c")
        s = lax.axis_index("s")
        g_local = s // _GPC
        quarter = s % _GPC
        col0 = (c * _GPC + g_local) * _GC
        row_base = quarter * _RPW
        lane = lax.iota(jnp.int32, 16)
        zeros16 = jnp.zeros((16,), jnp.int32)
        lanecols = [lane + np.int32(h * 16) for h in range(_NH)]
        nchunks = _RPW // _CHUNK
        max_r0 = np.int32(_N_ROWS - _CHUNK)

        def start(ci, buf, sem):
            r0 = jnp.minimum(row_base + ci * _CHUNK, max_r0)
            return pltpu.async_copy(
                x_hbm.at[pl.ds(r0, _CHUNK), pl.ds(col0, _GC)], buf, sem)

        def wait(buf, sem):
            pltpu.make_async_copy(
                x_hbm.at[pl.ds(0, _CHUNK), pl.ds(col0, _GC)], buf, sem
            ).wait()

        prefix = [[zeros16] * _NH, [zeros16] * _NH]
        target = [
            [jnp.full((16,), np.int32(_R1 + 1), jnp.int32)] * _NH,
            [jnp.full((16,), np.int32(_R2 + 1), jnp.int32)] * _NH,
        ]
        prefix = [list(t) for t in prefix]
        target = [list(t) for t in target]

        for p in range(4):
            shift = 24 - 8 * p

            @plsc.parallel_loop(0, 256, unroll=4)
            def _(b):
                for h in range(_NH):
                    hist[b, pl.ds(h * 16, 16)] = zeros16

            pref1 = list(prefix[0])
            pref2 = list(prefix[1])

            def process(buf, _p=p, _shift=shift, _p1=pref1, _p2=pref2):
                @plsc.parallel_loop(0, _CHUNK, unroll=2)
                def _(r):
                    for h in range(_NH):
                        v = buf[r, pl.ds(h * 16, 16)]
                        i32 = lax.bitcast_convert_type(v, jnp.int32)
                        sk = jnp.where(i32 < 0, i32 ^ _LOW31, i32)
                        ub = sk ^ _INT_MIN
                        bucket = (
                            lax.shift_right_logical(ub, np.int32(_shift))
                            & np.int32(0xFF))
                        if _p == 0:
                            val = jnp.full(
                                (16,), np.int32(0x10001), jnp.int32)
                        else:
                            hi = lax.shift_right_logical(
                                ub, np.int32(32 - 8 * _p))
                            val = (
                                jnp.where(hi == _p1[h],
                                          np.int32(1), np.int32(0))
                                + jnp.where(hi == _p2[h],
                                            np.int32(0x10000), np.int32(0)))
                        plsc.addupdate_scatter(
                            hist, [bucket, lanecols[h]], val)

            start(0, buf0, sem0)
            start(1, buf1, sem1)

            def stream_body(i, _):
                ci = i * 2
                wait(buf0, sem0)
                process(buf0)
                start(ci + 2, buf0, sem0)
                wait(buf1, sem1)
                process(buf1)
                start(ci + 3, buf1, sem1)
                return 0

            lax.fori_loop(0, nchunks // 2, stream_body, 0)
            wait(buf0, sem0)
            wait(buf1, sem1)

            pltpu.sync_copy(hist, shared.at[s])
            plsc.subcore_barrier()
            pltpu.sync_copy(shared.at[g_local * _GPC], hist)

            def merge_body(m, _):
                qq = m // 4 + 1
                part = m % 4
                partner = g_local * _GPC + qq
                pltpu.sync_copy(
                    shared.at[partner, pl.ds(part * 64, 64)], rb)

                @plsc.parallel_loop(0, 64, unroll=2)
                def _(b):
                    for h in range(_NH):
                        hist[part * 64 + b, pl.ds(h * 16, 16)] = (
                            hist[part * 64 + b, pl.ds(h * 16, 16)]
                            + rb[b, pl.ds(h * 16, 16)])
                return 0

            lax.fori_loop(0, 12, merge_body, 0)
            plsc.subcore_barrier()

            for hg in range(0, _NH, 4):
                hs = list(range(hg, hg + 4))

                def scan_body(b, carry, _hs=hs):
                    out = []
                    for j, h in enumerate(_hs):
                        (run1, sel1, below1, fnd1,
                         run2, sel2, below2, fnd2) = carry[j * 8:(j + 1) * 8]
                        cnt = hist[b, pl.ds(h * 16, 16)]
                        c1 = cnt & np.int32(0xFFFF)
                        c2 = lax.shift_right_logical(cnt, np.int32(16))
                        bvec = zeros16 + b

                        n1 = run1 + c1
                        cr1 = (fnd1 == 0) & (n1 >= target[0][h])
                        sel1 = jnp.where(cr1, bvec, sel1)
                        below1 = jnp.where(cr1, run1, below1)
                        fnd1 = jnp.where(cr1, zeros16 + 1, fnd1)

                        n2 = run2 + c2
                        cr2 = (fnd2 == 0) & (n2 >= target[1][h])
                        sel2 = jnp.where(cr2, bvec, sel2)
                        below2 = jnp.where(cr2, run2, below2)
                        fnd2 = jnp.where(cr2, zeros16 + 1, fnd2)
                        out.extend(
                            (n1, sel1, below1, fnd1,
                             n2, sel2, below2, fnd2))
                    return tuple(out)

                res = lax.fori_loop(0, 256, scan_body, (zeros16,) * 32)
                for j, h in enumerate(hs):
                    (_, sel1, below1, _, _, sel2, below2, _) = (
                        res[j * 8:(j + 1) * 8])
                    prefix[0][h] = (prefix[0][h] << 8) | sel1
                    target[0][h] = target[0][h] - below1
                    prefix[1][h] = (prefix[1][h] << 8) | sel2
                    target[1][h] = target[1][h] - below2

        @pl.when(quarter == 0)
        def _():
            for k, out_hbm in ((0, q1_hbm), (1, q2_hbm)):
                for h in range(_NH):
                    sk = prefix[k][h] ^ _INT_MIN
                    ib = jnp.where(sk < 0, sk ^ _LOW31, sk)
                    qbuf[pl.ds(h * 16, 16)] = lax.bitcast_convert_type(
                        ib, jnp.float32)
                pltpu.sync_copy(qbuf, out_hbm.at[pl.ds(col0, _GC)])

    return run(x)


def _tc_remap_body(x_ref, q1_ref, q2_ref, o_ref):
    x = x_ref[...]
    q1 = q1_ref[...].reshape(1, -1)
    q2 = q2_ref[...].reshape(1, -1)
    q0 = jnp.min(x, axis=0, keepdims=True)
    q3 = jnp.max(x, axis=0, keepdims=True)

    def scale(lo, hi):
        itv = hi - lo
        safe = jnp.where(itv == 0.0, np.float32(1.0), itv)
        return jnp.where(itv == 0.0, np.float32(0.0), _QLEN / safe)

    s0 = scale(q0, q1)
    s1 = scale(q1, q2)
    s2 = scale(q2, q3)
    out = jnp.where(
        x < q1,
        _OFFS[0] + x * s0,
        jnp.where(
            (x > q1) & (x < q2),
            _OFFS[1] + x * s1,
            jnp.where(x > q2, _OFFS[2] + x * s2, np.float32(0.0)),
        ),
    )
    o_ref[...] = out


def kernel(inputs):
    q1, q2 = _sc_quantiles(inputs)
    bc = 128
    out = pl.pallas_call(
        _tc_remap_body,
        grid=(_N_COLS // bc,),
        in_specs=[
            pl.BlockSpec((_N_ROWS, bc), lambda j: (0, j)),
            pl.BlockSpec((1, 1, bc), lambda j: (j, 0, 0)),
            pl.BlockSpec((1, 1, bc), lambda j: (j, 0, 0)),
        ],
        out_specs=pl.BlockSpec((_N_ROWS, bc), lambda j: (0, j)),
        out_shape=jax.ShapeDtypeStruct((_N_ROWS, _N_COLS), jnp.float32),
    )(inputs, q1.reshape(8, 1, bc), q2.reshape(8, 1, bc))
    return out


# speedup vs baseline: 3.5908x; 1.1961x over previous
import functools
import numpy as np
import jax
import jax.numpy as jnp
from jax import lax
from jax.experimental import pallas as pl
from jax.experimental.pallas import tpu as pltpu
from jax.experimental.pallas import tpu_sc as plsc

_N_ROWS = 16384
_N_COLS = 1024
_R1 = 5461
_R2 = 10922

_INT_MIN = np.int32(-(2**31))
_LOW31 = np.int32(2**31 - 1)
_BITS = [np.uint32(1 << b).astype(np.int32) for b in range(32)]

_QLEN = np.float32((1.0 - (-1.0)) / 3)
_OFFS = [np.float32(-1.0 + ((1.0 - (-1.0)) / 3) * i) for i in range(3)]


_SC_COL0 = 512
_SC_COLS = 512
_GPC = 2
_GC = 128
_PPG = 8
_RPW = _N_ROWS // _PPG
_CHUNK = 128
_NH = _GC // 16
_NS = 16


def _sc_quantiles(x):
    mesh = plsc.VectorSubcoreMesh(core_axis_name="c", subcore_axis_name="s")

    @functools.partial(
        pl.kernel,
        mesh=mesh,
        compiler_params=pltpu.CompilerParams(needs_layout_passes=False),
        out_type=[
            jax.ShapeDtypeStruct((_SC_COLS,), jnp.float32),
            jax.ShapeDtypeStruct((_SC_COLS,), jnp.float32),
        ],
        scratch_types=[
            pltpu.VMEM((_CHUNK, _GC), jnp.float32),
            pltpu.VMEM((_CHUNK, _GC), jnp.float32),
            pltpu.VMEM((256, _GC), jnp.int32),
            pltpu.VMEM((64, _GC), jnp.int32),
            pltpu.VMEM((_GC,), jnp.float32),
            pltpu.VMEM_SHARED((_NS, 256, _GC), jnp.int32),
            pltpu.SemaphoreType.DMA,
            pltpu.SemaphoreType.DMA,
        ],
    )
    def run(x_hbm, q1_hbm, q2_hbm, buf0, buf1, hist, rb, qbuf, shared,
            sem0, sem1):
        c = lax.axis_index("c")
        s = lax.axis_index("s")
        g_local = s // _PPG
        octant = s % _PPG
        col0 = _SC_COL0 + (c * _GPC + g_local) * _GC
        ocol0 = (c * _GPC + g_local) * _GC
        row_base = octant * _RPW
        lane = lax.iota(jnp.int32, 16)
        zeros16 = jnp.zeros((16,), jnp.int32)
        lanecols = [lane + np.int32(h * 16) for h in range(_NH)]
        nchunks = _RPW // _CHUNK
        max_r0 = np.int32(_N_ROWS - _CHUNK)

        def start(ci, buf, sem):
            r0 = jnp.minimum(row_base + ci * _CHUNK, max_r0)
            return pltpu.async_copy(
                x_hbm.at[pl.ds(r0, _CHUNK), pl.ds(col0, _GC)], buf, sem)

        def wait(buf, sem):
            pltpu.make_async_copy(
                x_hbm.at[pl.ds(0, _CHUNK), pl.ds(col0, _GC)], buf, sem
            ).wait()

        prefix = [[zeros16] * _NH, [zeros16] * _NH]
        target = [
            [jnp.full((16,), np.int32(_R1 + 1), jnp.int32)] * _NH,
            [jnp.full((16,), np.int32(_R2 + 1), jnp.int32)] * _NH,
        ]
        prefix = [list(t) for t in prefix]
        target = [list(t) for t in target]

        for p in range(4):
            shift = 24 - 8 * p

            @plsc.parallel_loop(0, 256, unroll=4)
            def _(b):
                for h in range(_NH):
                    hist[b, pl.ds(h * 16, 16)] = zeros16

            pref1 = list(prefix[0])
            pref2 = list(prefix[1])

            def process(buf, _p=p, _shift=shift, _p1=pref1, _p2=pref2):
                @plsc.parallel_loop(0, _CHUNK, unroll=2)
                def _(r):
                    for h in range(_NH):
                        v = buf[r, pl.ds(h * 16, 16)]
                        i32 = lax.bitcast_convert_type(v, jnp.int32)
                        sk = jnp.where(i32 < 0, i32 ^ _LOW31, i32)
                        ub = sk ^ _INT_MIN
                        bucket = (
                            lax.shift_right_logical(ub, np.int32(_shift))
                            & np.int32(0xFF))
                        if _p == 0:
                            val = jnp.full(
                                (16,), np.int32(0x10001), jnp.int32)
                        else:
                            hi = lax.shift_right_logical(
                                ub, np.int32(32 - 8 * _p))
                            val = (
                                jnp.where(hi == _p1[h],
                                          np.int32(1), np.int32(0))
                                + jnp.where(hi == _p2[h],
                                            np.int32(0x10000), np.int32(0)))
                        plsc.addupdate_scatter(
                            hist, [bucket, lanecols[h]], val)

            start(0, buf0, sem0)
            start(1, buf1, sem1)

            def stream_body(i, _):
                ci = i * 2
                wait(buf0, sem0)
                process(buf0)
                start(ci + 2, buf0, sem0)
                wait(buf1, sem1)
                process(buf1)
                start(ci + 3, buf1, sem1)
                return 0

            lax.fori_loop(0, nchunks // 2, stream_body, 0)
            wait(buf0, sem0)
            wait(buf1, sem1)

            pltpu.sync_copy(hist, shared.at[s])
            plsc.subcore_barrier()
            pltpu.sync_copy(shared.at[g_local * _PPG], hist)

            def merge_body(m, _):
                qq = m // 4 + 1
                part = m % 4
                partner = g_local * _PPG + qq
                pltpu.sync_copy(
                    shared.at[partner, pl.ds(part * 64, 64)], rb)

                @plsc.parallel_loop(0, 64, unroll=2)
                def _(b):
                    for h in range(_NH):
                        hist[part * 64 + b, pl.ds(h * 16, 16)] = (
                            hist[part * 64 + b, pl.ds(h * 16, 16)]
                            + rb[b, pl.ds(h * 16, 16)])
                return 0

            lax.fori_loop(0, (_PPG - 1) * 4, merge_body, 0)
            plsc.subcore_barrier()

            for hg in range(0, _NH, 4):
                hs = list(range(hg, hg + 4))

                def scan_body(b, carry, _hs=hs):
                    out = []
                    for j, h in enumerate(_hs):
                        (run1, sel1, below1, fnd1,
                         run2, sel2, below2, fnd2) = carry[j * 8:(j + 1) * 8]
                        cnt = hist[b, pl.ds(h * 16, 16)]
                        c1 = cnt & np.int32(0xFFFF)
                        c2 = lax.shift_right_logical(cnt, np.int32(16))
                        bvec = zeros16 + b

                        n1 = run1 + c1
                        cr1 = (fnd1 == 0) & (n1 >= target[0][h])
                        sel1 = jnp.where(cr1, bvec, sel1)
                        below1 = jnp.where(cr1, run1, below1)
                        fnd1 = jnp.where(cr1, zeros16 + 1, fnd1)

                        n2 = run2 + c2
                        cr2 = (fnd2 == 0) & (n2 >= target[1][h])
                        sel2 = jnp.where(cr2, bvec, sel2)
                        below2 = jnp.where(cr2, run2, below2)
                        fnd2 = jnp.where(cr2, zeros16 + 1, fnd2)
                        out.extend(
                            (n1, sel1, below1, fnd1,
                             n2, sel2, below2, fnd2))
                    return tuple(out)

                res = lax.fori_loop(0, 256, scan_body, (zeros16,) * 32)
                for j, h in enumerate(hs):
                    (_, sel1, below1, _, _, sel2, below2, _) = (
                        res[j * 8:(j + 1) * 8])
                    prefix[0][h] = (prefix[0][h] << 8) | sel1
                    target[0][h] = target[0][h] - below1
                    prefix[1][h] = (prefix[1][h] << 8) | sel2
                    target[1][h] = target[1][h] - below2

        @pl.when(octant == 0)
        def _():
            for k, out_hbm in ((0, q1_hbm), (1, q2_hbm)):
                for h in range(_NH):
                    sk = prefix[k][h] ^ _INT_MIN
                    ib = jnp.where(sk < 0, sk ^ _LOW31, sk)
                    qbuf[pl.ds(h * 16, 16)] = lax.bitcast_convert_type(
                        ib, jnp.float32)
                pltpu.sync_copy(qbuf, out_hbm.at[pl.ds(ocol0, _GC)])

    return run(x)


def _tc_search_body(x_ref, q1_ref, q2_ref):
    i32 = jax.lax.bitcast_convert_type(x_ref[...], jnp.int32)
    s = jnp.where(i32 < 0, i32 ^ _LOW31, i32)

    c = s.shape[1]
    ones = jnp.ones((1, _N_ROWS), jnp.bfloat16)

    def count_less(ts):
        m = (s < ts).astype(jnp.bfloat16)
        return jax.lax.dot_general(
            ones, m, (((1,), (0,)), ((), ())),
            preferred_element_type=jnp.float32)

    neg = count_less(jnp.zeros((1, c), jnp.int32))
    acc1 = jnp.where(neg >= np.float32(_R1 + 1),
                     jnp.zeros((1, c), jnp.int32), _BITS[31])
    acc2 = jnp.where(neg >= np.float32(_R2 + 1),
                     jnp.zeros((1, c), jnp.int32), _BITS[31])
    for b in range(30, -1, -1):
        bit = _BITS[b]
        t1 = acc1 | bit
        t2 = acc2 | bit
        cnt1 = count_less(t1 ^ _INT_MIN)
        cnt2 = count_less(t2 ^ _INT_MIN)
        acc1 = jnp.where(cnt1 >= np.float32(_R1 + 1), acc1, t1)
        acc2 = jnp.where(cnt2 >= np.float32(_R2 + 1), acc2, t2)

    def to_float(acc):
        sk = acc ^ _INT_MIN
        ib = jnp.where(sk < 0, sk ^ _LOW31, sk)
        return jax.lax.bitcast_convert_type(ib, jnp.float32)

    q1_ref[...] = to_float(acc1).reshape(1, 1, -1)
    q2_ref[...] = to_float(acc2).reshape(1, 1, -1)


def _tc_search(x):
    bc = 128
    return pl.pallas_call(
        _tc_search_body,
        grid=(_SC_COL0 // bc,),
        in_specs=[pl.BlockSpec((_N_ROWS, bc), lambda j: (0, j))],
        out_specs=[
            pl.BlockSpec((1, 1, bc), lambda j: (j, 0, 0)),
            pl.BlockSpec((1, 1, bc), lambda j: (j, 0, 0)),
        ],
        out_shape=[
            jax.ShapeDtypeStruct((_SC_COL0 // bc, 1, bc), jnp.float32),
            jax.ShapeDtypeStruct((_SC_COL0 // bc, 1, bc), jnp.float32),
        ],
        compiler_params=pltpu.CompilerParams(
            vmem_limit_bytes=64 * 1024 * 1024),
    )(x)


def _tc_remap_body(x_ref, q1_ref, q2_ref, o_ref):
    x = x_ref[...]
    q1 = q1_ref[...].reshape(1, -1)
    q2 = q2_ref[...].reshape(1, -1)
    q0 = jnp.min(x, axis=0, keepdims=True)
    q3 = jnp.max(x, axis=0, keepdims=True)

    def scale(lo, hi):
        itv = hi - lo
        safe = jnp.where(itv == 0.0, np.float32(1.0), itv)
        return jnp.where(itv == 0.0, np.float32(0.0), _QLEN / safe)

    s0 = scale(q0, q1)
    s1 = scale(q1, q2)
    s2 = scale(q2, q3)
    out = jnp.where(
        x < q1,
        _OFFS[0] + x * s0,
        jnp.where(
            (x > q1) & (x < q2),
            _OFFS[1] + x * s1,
            jnp.where(x > q2, _OFFS[2] + x * s2, np.float32(0.0)),
        ),
    )
    o_ref[...] = out


def kernel(inputs):
    bc = 128
    qt1, qt2 = _tc_search(inputs)
    qs1, qs2 = _sc_quantiles(inputs)
    q1 = jnp.concatenate([qt1, qs1.reshape(4, 1, bc)], axis=0)
    q2 = jnp.concatenate([qt2, qs2.reshape(4, 1, bc)], axis=0)
    out = pl.pallas_call(
        _tc_remap_body,
        grid=(_N_COLS // bc,),
        in_specs=[
            pl.BlockSpec((_N_ROWS, bc), lambda j: (0, j)),
            pl.BlockSpec((1, 1, bc), lambda j: (j, 0, 0)),
            pl.BlockSpec((1, 1, bc), lambda j: (j, 0, 0)),
        ],
        out_specs=pl.BlockSpec((_N_ROWS, bc), lambda j: (0, j)),
        out_shape=jax.ShapeDtypeStruct((_N_ROWS, _N_COLS), jnp.float32),
    )(inputs, q1, q2)
    return out


# speedup vs baseline: 4.8117x; 1.3400x over previous
import functools
import numpy as np
import jax
import jax.numpy as jnp
from jax import lax
from jax.experimental import pallas as pl
from jax.experimental.pallas import tpu as pltpu
from jax.experimental.pallas import tpu_sc as plsc

_N_ROWS = 16384
_N_COLS = 1024
_R1 = 5461
_R2 = 10922

_INT_MIN = np.int32(-(2**31))
_LOW31 = np.int32(2**31 - 1)
_BITS = [np.uint32(1 << b).astype(np.int32) for b in range(32)]

_QLEN = np.float32((1.0 - (-1.0)) / 3)
_OFFS = [np.float32(-1.0 + ((1.0 - (-1.0)) / 3) * i) for i in range(3)]


_SC_COL0 = 512
_SC_COLS = 512
_GPC = 2
_GC = 128
_PPG = 8
_RPW = _N_ROWS // _PPG
_CHUNK = 256
_NH = _GC // 16
_NS = 16


def _sc_quantiles(x):
    mesh = plsc.VectorSubcoreMesh(core_axis_name="c", subcore_axis_name="s")

    @functools.partial(
        pl.kernel,
        mesh=mesh,
        compiler_params=pltpu.CompilerParams(needs_layout_passes=False),
        out_type=[
            jax.ShapeDtypeStruct((_SC_COLS,), jnp.float32),
            jax.ShapeDtypeStruct((_SC_COLS,), jnp.float32),
        ],
        scratch_types=[
            pltpu.VMEM((_CHUNK, _GC), jnp.float32),
            pltpu.VMEM((_CHUNK, _GC), jnp.float32),
            pltpu.VMEM((256, _GC), jnp.int32),
            pltpu.VMEM((32, _GC), jnp.int32),
            pltpu.VMEM((_GC,), jnp.float32),
            pltpu.VMEM((2, 128), jnp.int32),
            pltpu.VMEM_SHARED((_GPC * 256, _GC), jnp.int32),
            pltpu.SemaphoreType.DMA,
            pltpu.SemaphoreType.DMA,
        ],
    )
    def run(x_hbm, q1_hbm, q2_hbm, buf0, buf1, hist, zsrc, qbuf, idxr,
            shared, sem0, sem1):
        c = lax.axis_index("c")
        s = lax.axis_index("s")
        g_local = s // _PPG
        octant = s % _PPG
        col0 = _SC_COL0 + (c * _GPC + g_local) * _GC
        ocol0 = (c * _GPC + g_local) * _GC
        row_base = octant * _RPW
        lane = lax.iota(jnp.int32, 16)
        zeros16 = jnp.zeros((16,), jnp.int32)
        lanecols = [lane + np.int32(h * 16) for h in range(_NH)]
        nchunks = _RPW // _CHUNK
        max_r0 = np.int32(_N_ROWS - _CHUNK)

        def start(ci, buf, sem):
            r0 = jnp.minimum(row_base + ci * _CHUNK, max_r0)
            return pltpu.async_copy(
                x_hbm.at[pl.ds(r0, _CHUNK), pl.ds(col0, _GC)], buf, sem)

        def wait(buf, sem):
            pltpu.make_async_copy(
                x_hbm.at[pl.ds(0, _CHUNK), pl.ds(col0, _GC)], buf, sem
            ).wait()

        prefix = [[zeros16] * _NH, [zeros16] * _NH]
        target = [
            [jnp.full((16,), np.int32(_R1 + 1), jnp.int32)] * _NH,
            [jnp.full((16,), np.int32(_R2 + 1), jnp.int32)] * _NH,
        ]
        prefix = [list(t) for t in prefix]
        target = [list(t) for t in target]

        for j in range(2):
            for i in range(8):
                idxr[j, pl.ds(i * 16, 16)] = (
                    lane + (g_local * 256 + np.int32(j * 128 + i * 16)))

        @plsc.parallel_loop(0, 32, unroll=4)
        def _(b):
            for h in range(_NH):
                zsrc[b, pl.ds(h * 16, 16)] = zeros16

        for p in range(4):
            shift = 24 - 8 * p

            pltpu.sync_copy(
                zsrc, shared.at[pl.ds(g_local * 256 + octant * 32, 32)])

            @plsc.parallel_loop(0, 256, unroll=4)
            def _(b):
                for h in range(_NH):
                    hist[b, pl.ds(h * 16, 16)] = zeros16

            plsc.subcore_barrier()

            pref1 = list(prefix[0])
            pref2 = list(prefix[1])

            def process(buf, _p=p, _shift=shift, _p1=pref1, _p2=pref2):
                @plsc.parallel_loop(0, _CHUNK, unroll=2)
                def _(r):
                    for h in range(_NH):
                        v = buf[r, pl.ds(h * 16, 16)]
                        i32 = lax.bitcast_convert_type(v, jnp.int32)
                        sk = jnp.where(i32 < 0, i32 ^ _LOW31, i32)
                        ub = sk ^ _INT_MIN
                        bucket = (
                            lax.shift_right_logical(ub, np.int32(_shift))
                            & np.int32(0xFF))
                        if _p == 0:
                            val = jnp.full(
                                (16,), np.int32(0x10001), jnp.int32)
                        else:
                            hi = lax.shift_right_logical(
                                ub, np.int32(32 - 8 * _p))
                            val = (
                                jnp.where(hi == _p1[h],
                                          np.int32(1), np.int32(0))
                                + jnp.where(hi == _p2[h],
                                            np.int32(0x10000), np.int32(0)))
                        plsc.addupdate_scatter(
                            hist, [bucket, lanecols[h]], val)

            start(0, buf0, sem0)
            start(1, buf1, sem1)

            def stream_body(i, _):
                ci = i * 2
                wait(buf0, sem0)
                process(buf0)
                start(ci + 2, buf0, sem0)
                wait(buf1, sem1)
                process(buf1)
                start(ci + 3, buf1, sem1)
                return 0

            lax.fori_loop(0, nchunks // 2, stream_body, 0)
            wait(buf0, sem0)
            wait(buf1, sem1)

            for j in range(2):
                pltpu.sync_copy(
                    hist.at[pl.ds(j * 128, 128)],
                    shared.at[idxr.at[j]], add=True)
            plsc.subcore_barrier()
            pltpu.sync_copy(shared.at[pl.ds(g_local * 256, 256)], hist)
            plsc.subcore_barrier()

            for hg in range(0, _NH, 4):
                hs = list(range(hg, hg + 4))

                def scan_body(b, carry, _hs=hs):
                    out = []
                    for j, h in enumerate(_hs):
                        (run1, sel1, below1, fnd1,
                         run2, sel2, below2, fnd2) = carry[j * 8:(j + 1) * 8]
                        cnt = hist[b, pl.ds(h * 16, 16)]
                        c1 = cnt & np.int32(0xFFFF)
                        c2 = lax.shift_right_logical(cnt, np.int32(16))
                        bvec = zeros16 + b

                        n1 = run1 + c1
                        cr1 = (fnd1 == 0) & (n1 >= target[0][h])
                        sel1 = jnp.where(cr1, bvec, sel1)
                        below1 = jnp.where(cr1, run1, below1)
                        fnd1 = jnp.where(cr1, zeros16 + 1, fnd1)

                        n2 = run2 + c2
                        cr2 = (fnd2 == 0) & (n2 >= target[1][h])
                        sel2 = jnp.where(cr2, bvec, sel2)
                        below2 = jnp.where(cr2, run2, below2)
                        fnd2 = jnp.where(cr2, zeros16 + 1, fnd2)
                        out.extend(
                            (n1, sel1, below1, fnd1,
                             n2, sel2, below2, fnd2))
                    return tuple(out)

                res = lax.fori_loop(0, 256, scan_body, (zeros16,) * 32)
                for j, h in enumerate(hs):
                    (_, sel1, below1, _, _, sel2, below2, _) = (
                        res[j * 8:(j + 1) * 8])
                    prefix[0][h] = (prefix[0][h] << 8) | sel1
                    target[0][h] = target[0][h] - below1
                    prefix[1][h] = (prefix[1][h] << 8) | sel2
                    target[1][h] = target[1][h] - below2

        @pl.when(octant == 0)
        def _():
            for k, out_hbm in ((0, q1_hbm), (1, q2_hbm)):
                for h in range(_NH):
                    sk = prefix[k][h] ^ _INT_MIN
                    ib = jnp.where(sk < 0, sk ^ _LOW31, sk)
                    qbuf[pl.ds(h * 16, 16)] = lax.bitcast_convert_type(
                        ib, jnp.float32)
                pltpu.sync_copy(qbuf, out_hbm.at[pl.ds(ocol0, _GC)])

    return run(x)


def _tc_search_body(x_ref, q1_ref, q2_ref):
    i32 = jax.lax.bitcast_convert_type(x_ref[...], jnp.int32)
    s = jnp.where(i32 < 0, i32 ^ _LOW31, i32)

    c = s.shape[1]
    ones = jnp.ones((1, _N_ROWS), jnp.bfloat16)

    def count_less(ts):
        m = (s < ts).astype(jnp.bfloat16)
        return jax.lax.dot_general(
            ones, m, (((1,), (0,)), ((), ())),
            preferred_element_type=jnp.float32)

    neg = count_less(jnp.zeros((1, c), jnp.int32))
    acc1 = jnp.where(neg >= np.float32(_R1 + 1),
                     jnp.zeros((1, c), jnp.int32), _BITS[31])
    acc2 = jnp.where(neg >= np.float32(_R2 + 1),
                     jnp.zeros((1, c), jnp.int32), _BITS[31])
    for b in range(30, -1, -1):
        bit = _BITS[b]
        t1 = acc1 | bit
        t2 = acc2 | bit
        cnt1 = count_less(t1 ^ _INT_MIN)
        cnt2 = count_less(t2 ^ _INT_MIN)
        acc1 = jnp.where(cnt1 >= np.float32(_R1 + 1), acc1, t1)
        acc2 = jnp.where(cnt2 >= np.float32(_R2 + 1), acc2, t2)

    def to_float(acc):
        sk = acc ^ _INT_MIN
        ib = jnp.where(sk < 0, sk ^ _LOW31, sk)
        return jax.lax.bitcast_convert_type(ib, jnp.float32)

    q1_ref[...] = to_float(acc1).reshape(1, 1, -1)
    q2_ref[...] = to_float(acc2).reshape(1, 1, -1)


def _tc_search(x):
    bc = 128
    return pl.pallas_call(
        _tc_search_body,
        grid=(_SC_COL0 // bc,),
        in_specs=[pl.BlockSpec((_N_ROWS, bc), lambda j: (0, j))],
        out_specs=[
            pl.BlockSpec((1, 1, bc), lambda j: (j, 0, 0)),
            pl.BlockSpec((1, 1, bc), lambda j: (j, 0, 0)),
        ],
        out_shape=[
            jax.ShapeDtypeStruct((_SC_COL0 // bc, 1, bc), jnp.float32),
            jax.ShapeDtypeStruct((_SC_COL0 // bc, 1, bc), jnp.float32),
        ],
        compiler_params=pltpu.CompilerParams(
            vmem_limit_bytes=64 * 1024 * 1024),
    )(x)


def _tc_remap_body(x_ref, q1_ref, q2_ref, o_ref):
    x = x_ref[...]
    q1 = q1_ref[...].reshape(1, -1)
    q2 = q2_ref[...].reshape(1, -1)
    q0 = jnp.min(x, axis=0, keepdims=True)
    q3 = jnp.max(x, axis=0, keepdims=True)

    def scale(lo, hi):
        itv = hi - lo
        safe = jnp.where(itv == 0.0, np.float32(1.0), itv)
        return jnp.where(itv == 0.0, np.float32(0.0), _QLEN / safe)

    s0 = scale(q0, q1)
    s1 = scale(q1, q2)
    s2 = scale(q2, q3)
    out = jnp.where(
        x < q1,
        _OFFS[0] + x * s0,
        jnp.where(
            (x > q1) & (x < q2),
            _OFFS[1] + x * s1,
            jnp.where(x > q2, _OFFS[2] + x * s2, np.float32(0.0)),
        ),
    )
    o_ref[...] = out


def kernel(inputs):
    bc = 128
    qt1, qt2 = _tc_search(inputs)
    qs1, qs2 = _sc_quantiles(inputs)
    q1 = jnp.concatenate([qt1, qs1.reshape(4, 1, bc)], axis=0)
    q2 = jnp.concatenate([qt2, qs2.reshape(4, 1, bc)], axis=0)
    out = pl.pallas_call(
        _tc_remap_body,
        grid=(_N_COLS // bc,),
        in_specs=[
            pl.BlockSpec((_N_ROWS, bc), lambda j: (0, j)),
            pl.BlockSpec((1, 1, bc), lambda j: (j, 0, 0)),
            pl.BlockSpec((1, 1, bc), lambda j: (j, 0, 0)),
        ],
        out_specs=pl.BlockSpec((_N_ROWS, bc), lambda j: (0, j)),
        out_shape=jax.ShapeDtypeStruct((_N_ROWS, _N_COLS), jnp.float32),
    )(inputs, q1, q2)
    return out


# speedup vs baseline: 5.1154x; 1.0631x over previous
import functools
import numpy as np
import jax
import jax.numpy as jnp
from jax import lax
from jax.experimental import pallas as pl
from jax.experimental.pallas import tpu as pltpu
from jax.experimental.pallas import tpu_sc as plsc

_N_ROWS = 16384
_N_COLS = 1024
_R1 = 5461
_R2 = 10922

_INT_MIN = np.int32(-(2**31))
_LOW31 = np.int32(2**31 - 1)
_BITS = [np.uint32(1 << b).astype(np.int32) for b in range(32)]

_QLEN = np.float32((1.0 - (-1.0)) / 3)
_OFFS = [np.float32(-1.0 + ((1.0 - (-1.0)) / 3) * i) for i in range(3)]


_SC_COL0 = 512
_SC_COLS = 512
_GPC = 2
_GC = 128
_PPG = 8
_RPW = _N_ROWS // _PPG
_CHUNK = 256
_NH = _GC // 16
_NS = 16


def _sc_quantiles(x):
    mesh = plsc.VectorSubcoreMesh(core_axis_name="c", subcore_axis_name="s")

    @functools.partial(
        pl.kernel,
        mesh=mesh,
        compiler_params=pltpu.CompilerParams(needs_layout_passes=False),
        out_type=[
            jax.ShapeDtypeStruct((_SC_COLS,), jnp.float32),
            jax.ShapeDtypeStruct((_SC_COLS,), jnp.float32),
        ],
        scratch_types=[
            pltpu.VMEM((_CHUNK, _GC), jnp.float32),
            pltpu.VMEM((_CHUNK, _GC), jnp.float32),
            pltpu.VMEM((256, _GC), jnp.int32),
            pltpu.VMEM((32, _GC), jnp.int32),
            pltpu.VMEM((_GC,), jnp.float32),
            pltpu.VMEM((2, 128), jnp.int32),
            pltpu.VMEM_SHARED((_GPC * 256, _GC), jnp.int32),
            pltpu.SemaphoreType.DMA,
            pltpu.SemaphoreType.DMA,
        ],
    )
    def run(x_hbm, q1_hbm, q2_hbm, buf0, buf1, hist, zsrc, qbuf, idxr,
            shared, sem0, sem1):
        c = lax.axis_index("c")
        s = lax.axis_index("s")
        g_local = s // _PPG
        octant = s % _PPG
        col0 = _SC_COL0 + (c * _GPC + g_local) * _GC
        ocol0 = (c * _GPC + g_local) * _GC
        row_base = octant * _RPW
        lane = lax.iota(jnp.int32, 16)
        zeros16 = jnp.zeros((16,), jnp.int32)
        lanecols = [lane + np.int32(h * 16) for h in range(_NH)]
        nchunks = _RPW // _CHUNK
        max_r0 = np.int32(_N_ROWS - _CHUNK)

        def start(ci, buf, sem):
            r0 = jnp.minimum(row_base + ci * _CHUNK, max_r0)
            return pltpu.async_copy(
                x_hbm.at[pl.ds(r0, _CHUNK), pl.ds(col0, _GC)], buf, sem)

        def wait(buf, sem):
            pltpu.make_async_copy(
                x_hbm.at[pl.ds(0, _CHUNK), pl.ds(col0, _GC)], buf, sem
            ).wait()

        prefix = [[zeros16] * _NH, [zeros16] * _NH]
        target = [
            [jnp.full((16,), np.int32(_R1 + 1), jnp.int32)] * _NH,
            [jnp.full((16,), np.int32(_R2 + 1), jnp.int32)] * _NH,
        ]
        prefix = [list(t) for t in prefix]
        target = [list(t) for t in target]

        for j in range(2):
            for i in range(8):
                idxr[j, pl.ds(i * 16, 16)] = (
                    lane + (g_local * 256 + np.int32(j * 128 + i * 16)))

        @plsc.parallel_loop(0, 32, unroll=4)
        def _(b):
            for h in range(_NH):
                zsrc[b, pl.ds(h * 16, 16)] = zeros16

        for p in range(4):
            shift = 24 - 8 * p

            pltpu.sync_copy(
                zsrc, shared.at[pl.ds(g_local * 256 + octant * 32, 32)])

            @plsc.parallel_loop(0, 256, unroll=4)
            def _(b):
                for h in range(_NH):
                    hist[b, pl.ds(h * 16, 16)] = zeros16

            plsc.subcore_barrier()

            pref1 = list(prefix[0])
            pref2 = list(prefix[1])

            def process(buf, _p=p, _shift=shift, _p1=pref1, _p2=pref2):
                @plsc.parallel_loop(0, _CHUNK, unroll=2)
                def _(r):
                    for h in range(_NH):
                        v = buf[r, pl.ds(h * 16, 16)]
                        i32 = lax.bitcast_convert_type(v, jnp.int32)
                        sk = jnp.where(i32 < 0, i32 ^ _LOW31, i32)
                        ub = sk ^ _INT_MIN
                        bucket = (
                            lax.shift_right_logical(ub, np.int32(_shift))
                            & np.int32(0xFF))
                        if _p == 0:
                            val = jnp.full(
                                (16,), np.int32(0x10001), jnp.int32)
                        else:
                            hi = lax.shift_right_logical(
                                ub, np.int32(32 - 8 * _p))
                            val = (
                                jnp.where(hi == _p1[h],
                                          np.int32(1), np.int32(0))
                                + jnp.where(hi == _p2[h],
                                            np.int32(0x10000), np.int32(0)))
                        plsc.addupdate_scatter(
                            hist, [bucket, lanecols[h]], val)

            start(0, buf0, sem0)
            start(1, buf1, sem1)

            def stream_body(i, _):
                ci = i * 2
                wait(buf0, sem0)
                process(buf0)
                start(ci + 2, buf0, sem0)
                wait(buf1, sem1)
                process(buf1)
                start(ci + 3, buf1, sem1)
                return 0

            lax.fori_loop(0, nchunks // 2, stream_body, 0)
            wait(buf0, sem0)
            wait(buf1, sem1)

            for j in range(2):
                pltpu.sync_copy(
                    hist.at[pl.ds(j * 128, 128)],
                    shared.at[idxr.at[j]], add=True)
            plsc.subcore_barrier()
            pltpu.sync_copy(shared.at[pl.ds(g_local * 256, 256)], hist)
            plsc.subcore_barrier()

            for hg in range(0, _NH, 4):
                hs = list(range(hg, hg + 4))

                def scan_body(b, carry, _hs=hs):
                    out = []
                    for j, h in enumerate(_hs):
                        (run1, sel1, below1, fnd1,
                         run2, sel2, below2, fnd2) = carry[j * 8:(j + 1) * 8]
                        cnt = hist[b, pl.ds(h * 16, 16)]
                        c1 = cnt & np.int32(0xFFFF)
                        c2 = lax.shift_right_logical(cnt, np.int32(16))
                        bvec = zeros16 + b

                        n1 = run1 + c1
                        cr1 = (fnd1 == 0) & (n1 >= target[0][h])
                        sel1 = jnp.where(cr1, bvec, sel1)
                        below1 = jnp.where(cr1, run1, below1)
                        fnd1 = jnp.where(cr1, zeros16 + 1, fnd1)

                        n2 = run2 + c2
                        cr2 = (fnd2 == 0) & (n2 >= target[1][h])
                        sel2 = jnp.where(cr2, bvec, sel2)
                        below2 = jnp.where(cr2, run2, below2)
                        fnd2 = jnp.where(cr2, zeros16 + 1, fnd2)
                        out.extend(
                            (n1, sel1, below1, fnd1,
                             n2, sel2, below2, fnd2))
                    return tuple(out)

                res = lax.fori_loop(0, 256, scan_body, (zeros16,) * 32)
                for j, h in enumerate(hs):
                    (_, sel1, below1, _, _, sel2, below2, _) = (
                        res[j * 8:(j + 1) * 8])
                    prefix[0][h] = (prefix[0][h] << 8) | sel1
                    target[0][h] = target[0][h] - below1
                    prefix[1][h] = (prefix[1][h] << 8) | sel2
                    target[1][h] = target[1][h] - below2

        @pl.when(octant == 0)
        def _():
            for k, out_hbm in ((0, q1_hbm), (1, q2_hbm)):
                for h in range(_NH):
                    sk = prefix[k][h] ^ _INT_MIN
                    ib = jnp.where(sk < 0, sk ^ _LOW31, sk)
                    qbuf[pl.ds(h * 16, 16)] = lax.bitcast_convert_type(
                        ib, jnp.float32)
                pltpu.sync_copy(qbuf, out_hbm.at[pl.ds(ocol0, _GC)])

    return run(x)


def _tc_search_remap_body(x_ref, o_ref):
    i32 = jax.lax.bitcast_convert_type(x_ref[...], jnp.int32)
    s = jnp.where(i32 < 0, i32 ^ _LOW31, i32)
    c = s.shape[1]
    ones = jnp.ones((1, _N_ROWS), jnp.bfloat16)

    def count_less(ts):
        m = (s < ts).astype(jnp.bfloat16)
        return jax.lax.dot_general(
            ones, m, (((1,), (0,)), ((), ())),
            preferred_element_type=jnp.float32)

    neg = count_less(jnp.zeros((1, c), jnp.int32))
    acc1 = jnp.where(neg >= np.float32(_R1 + 1),
                     jnp.zeros((1, c), jnp.int32), _BITS[31])
    acc2 = jnp.where(neg >= np.float32(_R2 + 1),
                     jnp.zeros((1, c), jnp.int32), _BITS[31])
    for b in range(30, -1, -1):
        bit = _BITS[b]
        t1 = acc1 | bit
        t2 = acc2 | bit
        cnt1 = count_less(t1 ^ _INT_MIN)
        cnt2 = count_less(t2 ^ _INT_MIN)
        acc1 = jnp.where(cnt1 >= np.float32(_R1 + 1), acc1, t1)
        acc2 = jnp.where(cnt2 >= np.float32(_R2 + 1), acc2, t2)

    def to_float(acc):
        sk = acc ^ _INT_MIN
        ib = jnp.where(sk < 0, sk ^ _LOW31, sk)
        return jax.lax.bitcast_convert_type(ib, jnp.float32)

    q1 = to_float(acc1)
    q2 = to_float(acc2)
    x = x_ref[...]
    q0 = jnp.min(x, axis=0, keepdims=True)
    q3 = jnp.max(x, axis=0, keepdims=True)

    def scale(lo, hi):
        itv = hi - lo
        safe = jnp.where(itv == 0.0, np.float32(1.0), itv)
        return jnp.where(itv == 0.0, np.float32(0.0), _QLEN / safe)

    s0 = scale(q0, q1)
    s1 = scale(q1, q2)
    s2 = scale(q2, q3)
    o_ref[...] = jnp.where(
        x < q1,
        _OFFS[0] + x * s0,
        jnp.where(
            (x > q1) & (x < q2),
            _OFFS[1] + x * s1,
            jnp.where(x > q2, _OFFS[2] + x * s2, np.float32(0.0)),
        ),
    )


def _tc_search_remap(x):
    bc = 128
    return pl.pallas_call(
        _tc_search_remap_body,
        grid=(_SC_COL0 // bc,),
        in_specs=[pl.BlockSpec((_N_ROWS, bc), lambda j: (0, j))],
        out_specs=pl.BlockSpec((_N_ROWS, bc), lambda j: (0, j)),
        out_shape=jax.ShapeDtypeStruct((_N_ROWS, _N_COLS), jnp.float32),
        compiler_params=pltpu.CompilerParams(
            vmem_limit_bytes=64 * 1024 * 1024),
    )(x)


def _tc_search_body(x_ref, q1_ref, q2_ref):
    i32 = jax.lax.bitcast_convert_type(x_ref[...], jnp.int32)
    s = jnp.where(i32 < 0, i32 ^ _LOW31, i32)

    c = s.shape[1]
    ones = jnp.ones((1, _N_ROWS), jnp.bfloat16)

    def count_less(ts):
        m = (s < ts).astype(jnp.bfloat16)
        return jax.lax.dot_general(
            ones, m, (((1,), (0,)), ((), ())),
            preferred_element_type=jnp.float32)

    neg = count_less(jnp.zeros((1, c), jnp.int32))
    acc1 = jnp.where(neg >= np.float32(_R1 + 1),
                     jnp.zeros((1, c), jnp.int32), _BITS[31])
    acc2 = jnp.where(neg >= np.float32(_R2 + 1),
                     jnp.zeros((1, c), jnp.int32), _BITS[31])
    for b in range(30, -1, -1):
        bit = _BITS[b]
        t1 = acc1 | bit
        t2 = acc2 | bit
        cnt1 = count_less(t1 ^ _INT_MIN)
        cnt2 = count_less(t2 ^ _INT_MIN)
        acc1 = jnp.where(cnt1 >= np.float32(_R1 + 1), acc1, t1)
        acc2 = jnp.where(cnt2 >= np.float32(_R2 + 1), acc2, t2)

    def to_float(acc):
        sk = acc ^ _INT_MIN
        ib = jnp.where(sk < 0, sk ^ _LOW31, sk)
        return jax.lax.bitcast_convert_type(ib, jnp.float32)

    q1_ref[...] = to_float(acc1).reshape(1, 1, -1)
    q2_ref[...] = to_float(acc2).reshape(1, 1, -1)


def _tc_search(x):
    bc = 128
    return pl.pallas_call(
        _tc_search_body,
        grid=(_SC_COL0 // bc,),
        in_specs=[pl.BlockSpec((_N_ROWS, bc), lambda j: (0, j))],
        out_specs=[
            pl.BlockSpec((1, 1, bc), lambda j: (j, 0, 0)),
            pl.BlockSpec((1, 1, bc), lambda j: (j, 0, 0)),
        ],
        out_shape=[
            jax.ShapeDtypeStruct((_SC_COL0 // bc, 1, bc), jnp.float32),
            jax.ShapeDtypeStruct((_SC_COL0 // bc, 1, bc), jnp.float32),
        ],
        compiler_params=pltpu.CompilerParams(
            vmem_limit_bytes=64 * 1024 * 1024),
    )(x)


def _tc_remap_body(x_ref, q1_ref, q2_ref, o_ref):
    x = x_ref[...]
    q1 = q1_ref[...].reshape(1, -1)
    q2 = q2_ref[...].reshape(1, -1)
    q0 = jnp.min(x, axis=0, keepdims=True)
    q3 = jnp.max(x, axis=0, keepdims=True)

    def scale(lo, hi):
        itv = hi - lo
        safe = jnp.where(itv == 0.0, np.float32(1.0), itv)
        return jnp.where(itv == 0.0, np.float32(0.0), _QLEN / safe)

    s0 = scale(q0, q1)
    s1 = scale(q1, q2)
    s2 = scale(q2, q3)
    out = jnp.where(
        x < q1,
        _OFFS[0] + x * s0,
        jnp.where(
            (x > q1) & (x < q2),
            _OFFS[1] + x * s1,
            jnp.where(x > q2, _OFFS[2] + x * s2, np.float32(0.0)),
        ),
    )
    o_ref[...] = out


def _tc_remap_b_body(x_ref, q1_ref, q2_ref, prev_ref, o_ref):
    del prev_ref
    _tc_remap_body(x_ref, q1_ref, q2_ref, o_ref)


def kernel(inputs):
    bc = 128
    out_a = _tc_search_remap(inputs)
    qs1, qs2 = _sc_quantiles(inputs)
    out = pl.pallas_call(
        _tc_remap_b_body,
        grid=(_SC_COLS // bc,),
        in_specs=[
            pl.BlockSpec((_N_ROWS, bc), lambda j: (0, j + 4)),
            pl.BlockSpec((1, 1, bc), lambda j: (j, 0, 0)),
            pl.BlockSpec((1, 1, bc), lambda j: (j, 0, 0)),
            pl.BlockSpec((8, bc), lambda j: (0, 0)),
        ],
        out_specs=pl.BlockSpec((_N_ROWS, bc), lambda j: (0, j + 4)),
        out_shape=jax.ShapeDtypeStruct((_N_ROWS, _N_COLS), jnp.float32),
        input_output_aliases={3: 0},
    )(inputs, qs1.reshape(4, 1, bc), qs2.reshape(4, 1, bc), out_a)
    return out
